# baseline (speedup 1.0000x reference)
import functools

import jax
import jax.numpy as jnp
from jax import lax
from jax.experimental import pallas as pl
from jax.experimental.pallas import tpu as pltpu
from jax.experimental.pallas import tpu_sc as plsc

VOCAB = 1000001
EMB_DIM = 64
XROWS = 4096
XCOLS = 200

_NC = 2
_NS = 16
_NW = _NC * _NS
_RPW = XROWS // _NW
_NBUF = 4
_SPLIT = 128


def _make_kernel():
    mesh = plsc.VectorSubcoreMesh(core_axis_name="c", subcore_axis_name="s")

    @functools.partial(
        pl.kernel,
        mesh=mesh,
        compiler_params=pltpu.CompilerParams(use_tc_tiling_on_sc=False),
        out_type=jax.ShapeDtypeStruct((XROWS, XCOLS, EMB_DIM), jnp.float32),
        scratch_types=[
            pltpu.VMEM((_RPW, XCOLS), jnp.int32),
            *[pltpu.VMEM((XCOLS, EMB_DIM), jnp.float32) for _ in range(_NBUF)],
            *[pltpu.SemaphoreType.DMA for _ in range(2 * _NBUF)],
        ],
    )
    def emb_kernel(x_hbm, table_hbm, out_hbm, idx_v,
                   r0, r1, r2, r3, g0, g1, g2, g3, s0, s1, s2, s3):
        rows = (r0, r1, r2, r3)
        gsem = (g0, g1, g2, g3)
        ssem = (s0, s1, s2, s3)

        wid = lax.axis_index("s") * _NC + lax.axis_index("c")
        base = pl.multiple_of(wid * _RPW, 8)
        pltpu.sync_copy(x_hbm.at[pl.ds(base, _RPW)], idx_v)

        def issue_gather(i, b):
            pltpu.async_copy(
                table_hbm.at[idx_v.at[i, pl.ds(0, _SPLIT)]],
                rows[b].at[pl.ds(0, _SPLIT)], gsem[b])
            pltpu.async_copy(
                table_hbm.at[idx_v.at[i, pl.ds(_SPLIT, XCOLS - _SPLIT)]],
                rows[b].at[pl.ds(_SPLIT, XCOLS - _SPLIT)], gsem[b])

        def wait_gather(b):
            pltpu.make_async_copy(out_hbm.at[0], rows[b], gsem[b]).wait()

        def issue_store(i, b):
            pltpu.async_copy(rows[b], out_hbm.at[base + i], ssem[b])

        def wait_store(b):
            pltpu.make_async_copy(rows[b], out_hbm.at[0], ssem[b]).wait()

        def compute(b):
            r = rows[b]

            def row_body(i, carry):
                for u in range(2):
                    for j in range(EMB_DIM // 16):
                        v = r[2 * i + u, pl.ds(j * 16, 16)]
                        t = jnp.exp(v + v)
                        r[2 * i + u, pl.ds(j * 16, 16)] = 1.0 - 2.0 / (t + 1.0)
                return carry

            lax.fori_loop(0, XCOLS // 2, row_body, 0)

        issue_gather(0, 0)
        issue_gather(1, 1)
        issue_gather(2, 2)

        wait_gather(0)
        compute(0)
        issue_store(0, 0)
        issue_gather(3, 3)

        for c in (1, 2, 3):
            b = c
            wait_gather(b)
            compute(b)
            issue_store(c, b)
            b2 = (b + 3) % _NBUF
            wait_store(b2)
            issue_gather(c + 3, b2)

        def group_body(g, carry):
            cbase = 4 * g + 4
            for b in range(_NBUF):
                c = cbase + b
                wait_gather(b)
                compute(b)
                issue_store(c, b)
                b2 = (b + 3) % _NBUF
                wait_store(b2)
                issue_gather(c + 3, b2)
            return carry

        lax.fori_loop(0, (_RPW - 8) // 4, group_body, 0)

        wait_gather(0)
        compute(0)
        issue_store(_RPW - 4, 0)
        wait_store(3)
        issue_gather(_RPW - 1, 3)

        for c in (_RPW - 3, _RPW - 2, _RPW - 1):
            b = c % _NBUF
            wait_gather(b)
            compute(b)
            issue_store(c, b)

        for b in range(_NBUF):
            wait_store(b)

    return emb_kernel


_EMB = _make_kernel()


def kernel(x, table):
    return _EMB(x.astype(jnp.int32), table)


# speedup vs baseline: 1.1668x; 1.1668x over previous
import functools

import jax
import jax.numpy as jnp
from jax import lax
from jax.experimental import pallas as pl
from jax.experimental.pallas import tpu as pltpu
from jax.experimental.pallas import tpu_sc as plsc

VOCAB = 1000001
EMB_DIM = 64
XROWS = 4096
XCOLS = 200

_NC = 2
_NS = 16
_NW = _NC * _NS
_B = XROWS * XCOLS
_C = 128
_NCHUNK = _B // _C // _NW
_NBUF = 3


def _make_kernel():
    mesh = plsc.VectorSubcoreMesh(core_axis_name="c", subcore_axis_name="s")

    @functools.partial(
        pl.kernel,
        mesh=mesh,
        compiler_params=pltpu.CompilerParams(use_tc_tiling_on_sc=True),
        out_type=jax.ShapeDtypeStruct((_B, EMB_DIM), jnp.float32),
        scratch_types=[
            pltpu.VMEM((_NCHUNK, _C), jnp.int32),
            *[pltpu.VMEM((_C, 2 * EMB_DIM), jnp.float32) for _ in range(_NBUF)],
            *[pltpu.VMEM((_C, EMB_DIM), jnp.float32) for _ in range(_NBUF)],
            *[pltpu.SemaphoreType.DMA for _ in range(2 * _NBUF)],
        ],
    )
    def emb_kernel(x_hbm, table_hbm, out_hbm, idx_v,
                   r0, r1, r2, c0, c1, c2, g0, g1, g2, s0, s1, s2):
        rows = (r0, r1, r2)
        cbuf = (c0, c1, c2)
        gsem = (g0, g1, g2)
        ssem = (s0, s1, s2)

        wid = lax.axis_index("s") * _NC + lax.axis_index("c")
        cbase = pl.multiple_of(wid * _NCHUNK, 8)
        pltpu.sync_copy(x_hbm.at[pl.ds(cbase, _NCHUNK)], idx_v)

        def issue_gather(c, b):
            pltpu.async_copy(table_hbm.at[idx_v.at[c]], rows[b], gsem[b])

        def wait_gather(b):
            pltpu.make_async_copy(
                table_hbm.at[pl.ds(0, _C)], rows[b], gsem[b]).wait()

        def issue_store(c, b):
            off = pl.multiple_of((cbase + c) * _C, 8)
            pltpu.async_copy(cbuf[b], out_hbm.at[pl.ds(off, _C)], ssem[b])

        def wait_store(b):
            pltpu.make_async_copy(
                cbuf[b], out_hbm.at[pl.ds(0, _C)], ssem[b]).wait()

        def compute(b):
            r = rows[b]
            cb = cbuf[b]

            def row_body(k2, carry):
                for u in range(2):
                    k = 2 * k2 + u
                    for j in range(EMB_DIM // 16):
                        val = r[k, pl.ds(j * 16, 16)]
                        t = jnp.exp(val + val)
                        cb[k, pl.ds(j * 16, 16)] = 1.0 - 2.0 / (t + 1.0)
                return carry

            lax.fori_loop(0, _C // 2, row_body, 0)

        issue_gather(0, 0)
        issue_gather(1, 1)

        wait_gather(0)
        compute(0)
        issue_store(0, 0)
        issue_gather(2, 2)

        for c in (1, 2):
            b = c
            wait_gather(b)
            compute(b)
            issue_store(c, b)
            b2 = (b + 2) % _NBUF
            wait_store(b2)
            issue_gather(c + 2, b2)

        def group_body(g, carry):
            c0_ = 3 * g + 3
            for b in range(_NBUF):
                c = c0_ + b
                wait_gather(b)
                compute(b)
                issue_store(c, b)
                b2 = (b + 2) % _NBUF
                wait_store(b2)
                issue_gather(c + 2, b2)
            return carry

        lax.fori_loop(0, (_NCHUNK - 5) // 3, group_body, 0)

        for c in (_NCHUNK - 2, _NCHUNK - 1):
            b = c % _NBUF
            wait_gather(b)
            compute(b)
            issue_store(c, b)

        for b in range(_NBUF):
            wait_store(b)

    return emb_kernel


_EMB = _make_kernel()


def kernel(x, table):
    t2 = jnp.concatenate([table, table], axis=1)
    x2 = jnp.reshape(x, (_B // _C, _C)).astype(jnp.int32)
    out = _EMB(x2, t2)
    return jnp.reshape(out, (XROWS, XCOLS, EMB_DIM))
